# baseline (speedup 1.0000x reference)
import jax
import jax.numpy as jnp
from jax import lax
from jax.experimental import pallas as pl
from jax.experimental.pallas import tpu as pltpu
from jax.experimental.pallas import tpu_sc as plsc

_NUM_CORES = 2
_NUM_SUBCORES = 16
_NW = _NUM_CORES * _NUM_SUBCORES
_B = 16384
_D = 16
_BPW = _B // _NW
_CHUNK = 128
_NCHUNK = _BPW // _CHUNK
_LAMBDA_C = 0.003


def _cl_body(feat_hbm, lbl_hbm, centers_hbm, out_hbm, idx_v, feat_v, rows_v,
             acc_v, sem):
    wid = lax.axis_index("s") * _NUM_CORES + lax.axis_index("c")
    base = wid * _BPW
    pltpu.sync_copy(lbl_hbm.at[pl.ds(wid * _NCHUNK, _NCHUNK)], idx_v)
    copies = [
        pltpu.async_copy(centers_hbm.at[idx_v.at[k]],
                         rows_v.at[pl.ds(k * _CHUNK, _CHUNK)], sem)
        for k in range(_NCHUNK)
    ]
    pltpu.sync_copy(feat_hbm.at[pl.ds(base * _D, _BPW * _D)], feat_v)
    for cp in copies:
        cp.wait()

    def step(i, acc):
        f = feat_v[pl.ds(i * _D, _D)]
        c = rows_v[i]
        d = f - c
        return acc + d * d

    acc = lax.fori_loop(0, _BPW, step, jnp.zeros((_D,), jnp.float32))
    acc_v[...] = acc
    pltpu.sync_copy(acc_v, out_hbm.at[wid])


@jax.jit
def kernel(features, labels, centers):
    lbl = labels.reshape(_B).astype(jnp.int32).reshape(_NW * _NCHUNK, _CHUNK)
    feat = features.reshape(_B * _D)
    mesh = plsc.VectorSubcoreMesh(core_axis_name="c", subcore_axis_name="s")
    partials = pl.kernel(
        _cl_body,
        out_type=jax.ShapeDtypeStruct((_NW, _D), jnp.float32),
        mesh=mesh,
        scratch_types=[
            pltpu.VMEM((_NCHUNK, _CHUNK), jnp.int32),
            pltpu.VMEM((_BPW * _D,), jnp.float32),
            pltpu.VMEM((_BPW, _D), jnp.float32),
            pltpu.VMEM((_D,), jnp.float32),
            pltpu.SemaphoreType.DMA,
        ],
        compiler_params=pltpu.CompilerParams(use_tc_tiling_on_sc=False),
    )(feat, lbl, centers)
    return _LAMBDA_C * (jnp.sum(partials) / 2.0 / _B)
